# baseline (speedup 1.0000x reference)
import jax
import jax.numpy as jnp
from jax.experimental import pallas as pl
from jax.experimental.pallas import tpu as pltpu

_COMMITMENT = 0.25
_EPS = 1e-12


def _wprep_kernel(w_ref, wa_ref):
    w = w_ref[...]
    wn2 = jnp.sum(w * w, axis=1, keepdims=True)
    wn = w / jnp.maximum(jnp.sqrt(wn2), _EPS)
    wsq = jnp.sum(wn * wn, axis=1, keepdims=True)
    wa_ref[...] = jnp.concatenate(
        [-2.0 * wn, wsq, jnp.ones_like(wsq)], axis=1)


def _vq_tile_kernel(x_ref, w_ref, q_ref, loss_ref, idx_ref):
    wa = w_ref[...]

    x = x_ref[0]
    xn2 = jnp.sum(x * x, axis=0, keepdims=True)
    xn = x / jnp.maximum(jnp.sqrt(xn2), _EPS)
    xsq = jnp.sum(xn * xn, axis=0, keepdims=True)
    xa = jnp.concatenate([xn, jnp.ones_like(xsq), xsq], axis=0)

    dist = jnp.dot(wa, xa, preferred_element_type=jnp.float32)

    idx0 = jnp.argmin(dist, axis=0)

    onehot = (jax.lax.broadcasted_iota(jnp.int32, dist.shape, 0)
              == idx0[None, :]).astype(jnp.float32)
    q = -0.5 * jax.lax.dot_general(wa[:, :64], onehot,
                                   dimension_numbers=(((0,), (0,)), ((), ())),
                                   preferred_element_type=jnp.float32)

    d = q - xn
    sq = d * d
    loss = jnp.mean(sq + _COMMITMENT * sq, axis=0)

    q_ref[0] = q
    loss_ref[0, 0, :] = loss
    idx_ref[0, 0, :] = (idx0 + 1).astype(jnp.int32)


def kernel(inputs, W):
    B, C, nz, nt, nr = inputs.shape
    S = nz * nt * nr
    x3 = inputs.reshape(B, C, S)
    W1 = W[1:]
    K = W1.shape[0]

    wa = pl.pallas_call(
        _wprep_kernel,
        out_shape=jax.ShapeDtypeStruct((K, C + 2), jnp.float32),
    )(W1)

    T = 4096
    grid = (B, S // T)

    q3, loss3, idx3 = pl.pallas_call(
        _vq_tile_kernel,
        grid=grid,
        in_specs=[
            pl.BlockSpec((1, C, T), lambda b, t: (b, 0, t)),
            pl.BlockSpec((K, C + 2), lambda b, t: (0, 0)),
        ],
        out_specs=[
            pl.BlockSpec((1, C, T), lambda b, t: (b, 0, t)),
            pl.BlockSpec((1, 1, T), lambda b, t: (b, 0, t)),
            pl.BlockSpec((1, 1, T), lambda b, t: (b, 0, t)),
        ],
        out_shape=[
            jax.ShapeDtypeStruct((B, C, S), jnp.float32),
            jax.ShapeDtypeStruct((B, 1, S), jnp.float32),
            jax.ShapeDtypeStruct((B, 1, S), jnp.int32),
        ],
        compiler_params=pltpu.CompilerParams(
            dimension_semantics=("parallel", "parallel")),
    )(x3, wa)

    quantized_out = q3.reshape(B, C, nz, nt, nr)
    vq_loss_spatial = loss3.reshape(B, nz, nt, nr)
    spatial_indices = idx3.reshape(B, nz, nt, nr)
    return quantized_out, vq_loss_spatial, spatial_indices


# speedup vs baseline: 1.0142x; 1.0142x over previous
import jax
import jax.numpy as jnp
from jax.experimental import pallas as pl
from jax.experimental.pallas import tpu as pltpu

_COMMITMENT = 0.25
_EPS = 1e-12


def _wprep_kernel(w_ref, wa_ref):
    w = w_ref[...]
    wn2 = jnp.sum(w * w, axis=1, keepdims=True)
    wn = w / jnp.maximum(jnp.sqrt(wn2), _EPS)
    wsq = jnp.sum(wn * wn, axis=1, keepdims=True)
    wa_ref[...] = jnp.concatenate(
        [-2.0 * wn, wsq, jnp.ones_like(wsq)], axis=1)


def _vq_tile_kernel(x_ref, w_ref, q_ref, loss_ref, idx_ref):
    wa = w_ref[...]

    x = x_ref[0]
    xn2 = jnp.sum(x * x, axis=0, keepdims=True)
    xn = x / jnp.maximum(jnp.sqrt(xn2), _EPS)
    xsq = jnp.sum(xn * xn, axis=0, keepdims=True)
    xa = jnp.concatenate([xn, jnp.ones_like(xsq), xsq], axis=0)

    dist = jnp.dot(wa, xa, preferred_element_type=jnp.float32)

    idx0 = jnp.argmin(dist, axis=0)

    onehot = (jax.lax.broadcasted_iota(jnp.int32, dist.shape, 0)
              == idx0[None, :]).astype(jnp.float32)
    q = -0.5 * jax.lax.dot_general(wa[:, :64], onehot,
                                   dimension_numbers=(((0,), (0,)), ((), ())),
                                   preferred_element_type=jnp.float32)

    d = q - xn
    sq = d * d
    loss = jnp.mean(sq + _COMMITMENT * sq, axis=0)

    q_ref[0] = q
    loss_ref[0, 0, :] = loss
    idx_ref[0, 0, :] = (idx0 + 1).astype(jnp.int32)


def kernel(inputs, W):
    B, C, nz, nt, nr = inputs.shape
    S = nz * nt * nr
    x3 = inputs.reshape(B, C, S)
    W1 = W[1:]
    K = W1.shape[0]

    wa = pl.pallas_call(
        _wprep_kernel,
        out_shape=jax.ShapeDtypeStruct((K, C + 2), jnp.float32),
    )(W1)

    T = 8192
    grid = (B, S // T)

    q3, loss3, idx3 = pl.pallas_call(
        _vq_tile_kernel,
        grid=grid,
        in_specs=[
            pl.BlockSpec((1, C, T), lambda b, t: (b, 0, t)),
            pl.BlockSpec((K, C + 2), lambda b, t: (0, 0)),
        ],
        out_specs=[
            pl.BlockSpec((1, C, T), lambda b, t: (b, 0, t)),
            pl.BlockSpec((1, 1, T), lambda b, t: (b, 0, t)),
            pl.BlockSpec((1, 1, T), lambda b, t: (b, 0, t)),
        ],
        out_shape=[
            jax.ShapeDtypeStruct((B, C, S), jnp.float32),
            jax.ShapeDtypeStruct((B, 1, S), jnp.float32),
            jax.ShapeDtypeStruct((B, 1, S), jnp.int32),
        ],
        compiler_params=pltpu.CompilerParams(
            dimension_semantics=("parallel", "parallel")),
    )(x3, wa)

    quantized_out = q3.reshape(B, C, nz, nt, nr)
    vq_loss_spatial = loss3.reshape(B, nz, nt, nr)
    spatial_indices = idx3.reshape(B, nz, nt, nr)
    return quantized_out, vq_loss_spatial, spatial_indices
